# baseline (speedup 1.0000x reference)
import functools

import jax
import jax.numpy as jnp
from jax.experimental import pallas as pl

TEMP = 1.0
LB_WEIGHT = 0.01


def _routing_body(x_ref, gw_ref, comb_ref, aux_ref):
    E = gw_ref.shape[0]
    T = x_ref.shape[0]
    logits_t = jax.lax.dot_general(
        gw_ref[...], x_ref[...],
        dimension_numbers=(((1,), (1,)), ((), ())),
        preferred_element_type=jnp.float32)
    logits_t = logits_t / TEMP
    mx = jnp.max(logits_t, axis=0, keepdims=True)
    ex = jnp.exp(logits_t - mx)
    p = ex / jnp.sum(ex, axis=0, keepdims=True)
    eidx = jax.lax.broadcasted_iota(jnp.int32, (E, T), 0)
    big = jnp.int32(E)
    p1 = jnp.max(p, axis=0, keepdims=True)
    i1 = jnp.min(jnp.where(p == p1, eidx, big), axis=0, keepdims=True)
    mask1 = eidx == i1
    pm = jnp.where(mask1, -1.0, p)
    p2 = jnp.max(pm, axis=0, keepdims=True)
    i2 = jnp.min(jnp.where(pm == p2, eidx, big), axis=0, keepdims=True)
    mask2 = eidx == i2
    norm = p1 + p2 + 1e-9
    comb_ref[:, 0, :] = (jnp.where(mask1, p1 / norm, 0.0)
                         + jnp.where(mask2, p2 / norm, 0.0))
    one_hot = (mask1 | mask2).astype(jnp.float32)
    frac = jnp.sum(one_hot, axis=1, keepdims=True) / (T * 2)
    meanp = jnp.mean(p, axis=1, keepdims=True)
    aux_ref[...] = jnp.sum(frac * meanp).reshape(1, 1) * (LB_WEIGHT * E)


def _expert_body(x_ref, wg_ref, wu_ref, wd_ref, comb_ref, out_ref, *, bm):
    e = pl.program_id(0)
    j = pl.program_id(1)
    m = pl.program_id(2)

    @pl.when((e == 0) & (j == 0))
    def _zero():
        out_ref[pl.ds(m * bm, bm), :] = jnp.zeros(
            (bm, out_ref.shape[1]), jnp.float32)

    xb = x_ref[...].astype(jnp.bfloat16)
    wg = wg_ref[0].astype(jnp.bfloat16)
    wu = wu_ref[0].astype(jnp.bfloat16)
    g = jax.lax.dot_general(xb, wg, (((1,), (1,)), ((), ())),
                            preferred_element_type=jnp.float32)
    u = jax.lax.dot_general(xb, wu, (((1,), (1,)), ((), ())),
                            preferred_element_type=jnp.float32)
    h = (g * jax.lax.logistic(g) * u).astype(jnp.bfloat16)
    wd = wd_ref[0].astype(jnp.bfloat16)
    y = jax.lax.dot_general(h, wd, (((1,), (1,)), ((), ())),
                            preferred_element_type=jnp.float32)
    c = comb_ref[0, 0, :]
    out_ref[pl.ds(m * bm, bm), :] += y * c[:, None]


def kernel(x, gate_w, gate_up_w, down_w):
    B, S, D = x.shape
    T = B * S
    E = gate_w.shape[0]
    DFF = down_w.shape[2]
    x_flat = x.reshape(T, D)

    comb_t, aux = pl.pallas_call(
        _routing_body,
        out_shape=(
            jax.ShapeDtypeStruct((E, 1, T), jnp.float32),
            jax.ShapeDtypeStruct((1, 1), jnp.float32),
        ),
    )(x_flat, gate_w)

    BM = 256 if T % 256 == 0 else T
    BN = 512 if DFF % 512 == 0 else DFF
    M = T // BM
    J = DFF // BN
    JG = DFF // BN

    out = pl.pallas_call(
        functools.partial(_expert_body, bm=BM),
        grid=(E, J, M),
        in_specs=[
            pl.BlockSpec((BM, D), lambda e, j, m: (m, 0)),
            pl.BlockSpec((1, BN, D), lambda e, j, m: (e, j, 0)),
            pl.BlockSpec((1, BN, D), lambda e, j, m, jg=JG: (e, jg + j, 0)),
            pl.BlockSpec((1, D, BN), lambda e, j, m: (e, 0, j)),
            pl.BlockSpec((1, 1, BM), lambda e, j, m: (e, 0, m)),
        ],
        out_specs=pl.BlockSpec((T, D), lambda e, j, m: (0, 0)),
        out_shape=jax.ShapeDtypeStruct((T, D), jnp.float32),
    )(x_flat, gate_up_w, gate_up_w, down_w, comb_t)

    return out.reshape(B, S, D), aux.reshape(())


# speedup vs baseline: 1.3470x; 1.3470x over previous
import functools

import jax
import jax.numpy as jnp
from jax import lax
from jax.experimental import pallas as pl
from jax.experimental.pallas import tpu as pltpu
from jax.experimental.pallas import tpu_sc as plsc

TEMP = 1.0
LB_WEIGHT = 0.01

NC = 2
NS = 16
NW = NC * NS


def _routing_body(x_ref, gw_ref, pos1_ref, pos2_ref, w1_ref, w2_ref,
                  used_ref, be_ref, aux_ref, *, bm, nm):
    E = gw_ref.shape[0]
    T = x_ref.shape[0]
    logits = lax.dot_general(
        x_ref[...], gw_ref[...],
        dimension_numbers=(((1,), (1,)), ((), ())),
        preferred_element_type=jnp.float32) / TEMP
    mx = jnp.max(logits, axis=1, keepdims=True)
    ex = jnp.exp(logits - mx)
    p = ex / jnp.sum(ex, axis=1, keepdims=True)
    eidx = lax.broadcasted_iota(jnp.int32, (T, E), 1)
    big = jnp.int32(E)
    p1 = jnp.max(p, axis=1, keepdims=True)
    i1 = jnp.min(jnp.where(p == p1, eidx, big), axis=1, keepdims=True)
    mask1 = eidx == i1
    pm = jnp.where(mask1, -1.0, p)
    p2 = jnp.max(pm, axis=1, keepdims=True)
    i2 = jnp.min(jnp.where(pm == p2, eidx, big), axis=1, keepdims=True)
    mask2 = eidx == i2
    norm = p1 + p2 + 1e-9
    w1_ref[...] = jnp.broadcast_to(p1 / norm, (T, 16))
    w2_ref[...] = jnp.broadcast_to(p2 / norm, (T, 16))

    oh = (mask1 | mask2).astype(jnp.float32)
    frac = jnp.sum(oh, axis=0, keepdims=True) / (T * 2)
    meanp = jnp.mean(p, axis=0, keepdims=True)
    aux_ref[...] = jnp.sum(frac * meanp).reshape(1, 1) * (LB_WEIGHT * E)

    CH = 512
    rr = lax.broadcasted_iota(jnp.int32, (CH, CH), 0)
    cc = lax.broadcasted_iota(jnp.int32, (CH, CH), 1)
    tril = (cc < rr).astype(jnp.bfloat16)
    pieces = []
    carry = jnp.zeros((1, E), jnp.float32)
    for c in range(T // CH):
        ohc = oh[c * CH:(c + 1) * CH, :]
        local = lax.dot_general(
            tril, ohc.astype(jnp.bfloat16),
            dimension_numbers=(((1,), (0,)), ((), ())),
            preferred_element_type=jnp.float32)
        pieces.append(local + carry)
        carry = carry + jnp.sum(ohc, axis=0, keepdims=True)
    cnt_before = jnp.concatenate(pieces, axis=0)
    counts = carry
    pc = jnp.ceil(counts / bm) * bm
    trE = (lax.broadcasted_iota(jnp.int32, (E, E), 0)
           < lax.broadcasted_iota(jnp.int32, (E, E), 1)).astype(jnp.float32)
    pad_off = lax.dot_general(
        pc, trE, dimension_numbers=(((1,), (0,)), ((), ())),
        preferred_element_type=jnp.float32)
    base1 = jnp.sum(jnp.where(mask1, pad_off, 0.0), axis=1, keepdims=True)
    base2 = jnp.sum(jnp.where(mask2, pad_off, 0.0), axis=1, keepdims=True)
    rank1 = jnp.sum(jnp.where(mask1, cnt_before, 0.0), axis=1, keepdims=True)
    rank2 = jnp.sum(jnp.where(mask2, cnt_before, 0.0), axis=1, keepdims=True)
    pos1_ref[...] = (base1 + rank1).astype(jnp.int32)
    pos2_ref[...] = (base2 + rank2).astype(jnp.int32)

    ends = pad_off + pc
    used_ref[...] = (jnp.sum(pc, axis=1, keepdims=True) / bm).astype(jnp.int32)
    iE = (lax.broadcasted_iota(jnp.int32, (E, E), 0)
          == lax.broadcasted_iota(jnp.int32, (E, E), 1)).astype(jnp.float32)
    ends_t = lax.dot_general(
        iE, ends, dimension_numbers=(((1,), (1,)), ((), ())),
        preferred_element_type=jnp.float32)
    bidx = lax.broadcasted_iota(jnp.int32, (1, nm), 1).astype(jnp.float32) * bm
    be = jnp.sum((bidx >= ends_t).astype(jnp.int32), axis=0, keepdims=True)
    be_ref[...] = jnp.minimum(be, E - 1)


def _sc_scatter_body(x_hbm, pos1_hbm, pos2_hbm, xs_hbm, rows_v, idx_v, sem):
    wid = lax.axis_index("s") * NC + lax.axis_index("c")
    T = x_hbm.shape[0]
    tpw = T // NW
    base = wid * tpw
    pltpu.sync_copy(x_hbm.at[pl.ds(base, tpw)], rows_v)
    pltpu.sync_copy(pos1_hbm.at[pl.ds(base, tpw)], idx_v)
    pltpu.async_copy(rows_v, xs_hbm.at[idx_v], sem).wait()
    pltpu.sync_copy(pos2_hbm.at[pl.ds(base, tpw)], idx_v)
    pltpu.async_copy(rows_v, xs_hbm.at[idx_v], sem).wait()


def _gemm_body(used_ref, be_ref, xs_ref, wg_ref, wu_ref, wd_ref, out_ref,
               *, bm):
    j = pl.program_id(0)
    m = pl.program_id(1)

    @pl.when(m < used_ref[0])
    def _():
        xb = xs_ref[...].astype(jnp.bfloat16)
        wg = wg_ref[0].astype(jnp.bfloat16)
        wu = wu_ref[0].astype(jnp.bfloat16)
        g = lax.dot_general(xb, wg, (((1,), (1,)), ((), ())),
                            preferred_element_type=jnp.float32)
        u = lax.dot_general(xb, wu, (((1,), (1,)), ((), ())),
                            preferred_element_type=jnp.float32)
        h = (g * lax.logistic(g) * u).astype(jnp.bfloat16)
        wd = wd_ref[0].astype(jnp.bfloat16)
        y = lax.dot_general(h, wd, (((1,), (1,)), ((), ())),
                            preferred_element_type=jnp.float32)

        @pl.when(j == 0)
        def _store():
            out_ref[pl.ds(m * bm, bm), :] = y

        @pl.when(j > 0)
        def _accum():
            out_ref[pl.ds(m * bm, bm), :] += y


def _sc_combine_body(ys_hbm, pos1_hbm, pos2_hbm, w1_hbm, w2_hbm, out_hbm,
                     a_v, b_v, idx_v, w1_v, w2_v, sem):
    wid = lax.axis_index("s") * NC + lax.axis_index("c")
    T = out_hbm.shape[0]
    D = out_hbm.shape[1]
    tpw = T // NW
    CHT = 32
    for ci in range(tpw // CHT):
        base = wid * tpw + ci * CHT
        pltpu.sync_copy(pos1_hbm.at[pl.ds(base, CHT)], idx_v)
        pltpu.async_copy(ys_hbm.at[idx_v], a_v, sem).wait()
        pltpu.sync_copy(pos2_hbm.at[pl.ds(base, CHT)], idx_v)
        pltpu.async_copy(ys_hbm.at[idx_v], b_v, sem).wait()
        pltpu.sync_copy(w1_hbm.at[pl.ds(base, CHT)], w1_v)
        pltpu.sync_copy(w2_hbm.at[pl.ds(base, CHT)], w2_v)

        def row_body(r, _):
            s1 = w1_v[r, :]
            s2 = w2_v[r, :]
            for cj in range(D // 16):
                sl = pl.ds(cj * 16, 16)
                a_v[r, sl] = a_v[r, sl] * s1 + b_v[r, sl] * s2
            return 0

        lax.fori_loop(0, CHT, row_body, 0)
        pltpu.sync_copy(a_v, out_hbm.at[pl.ds(base, CHT)])


def kernel(x, gate_w, gate_up_w, down_w):
    B, S, D = x.shape
    T = B * S
    E = gate_w.shape[0]
    DFF = down_w.shape[2]
    K = 2
    BM = 128
    BN = 512
    NM = (T * K) // BM + E
    MPAD = NM * BM
    J = DFF // BN
    JG = DFF // BN
    x_flat = x.reshape(T, D)

    pos1, pos2, w1, w2, used, be, aux = pl.pallas_call(
        functools.partial(_routing_body, bm=BM, nm=NM),
        out_shape=(
            jax.ShapeDtypeStruct((T, 1), jnp.int32),
            jax.ShapeDtypeStruct((T, 1), jnp.int32),
            jax.ShapeDtypeStruct((T, 16), jnp.float32),
            jax.ShapeDtypeStruct((T, 16), jnp.float32),
            jax.ShapeDtypeStruct((1, 1), jnp.int32),
            jax.ShapeDtypeStruct((1, NM), jnp.int32),
            jax.ShapeDtypeStruct((1, 1), jnp.float32),
        ),
    )(x_flat, gate_w)
    pos1 = pos1.reshape(T)
    pos2 = pos2.reshape(T)

    mesh = plsc.VectorSubcoreMesh(core_axis_name="c", subcore_axis_name="s")
    xs = pl.kernel(
        _sc_scatter_body,
        out_type=jax.ShapeDtypeStruct((MPAD, D), jnp.float32),
        mesh=mesh,
        scratch_types=[
            pltpu.VMEM((T // NW, D), jnp.float32),
            pltpu.VMEM((T // NW,), jnp.int32),
            pltpu.SemaphoreType.DMA,
        ],
    )(x_flat, pos1, pos2)

    ys = pl.pallas_call(
        functools.partial(_gemm_body, bm=BM),
        grid_spec=pltpu.PrefetchScalarGridSpec(
            num_scalar_prefetch=2,
            grid=(J, NM),
            in_specs=[
                pl.BlockSpec((BM, D), lambda j, m, u, b: (m, 0)),
                pl.BlockSpec((1, BN, D), lambda j, m, u, b: (b[m], j, 0)),
                pl.BlockSpec((1, BN, D),
                             lambda j, m, u, b, jg=JG: (b[m], jg + j, 0)),
                pl.BlockSpec((1, D, BN), lambda j, m, u, b: (b[m], 0, j)),
            ],
            out_specs=pl.BlockSpec((MPAD, D), lambda j, m, u, b: (0, 0)),
        ),
        out_shape=jax.ShapeDtypeStruct((MPAD, D), jnp.float32),
    )(used.reshape(1), be.reshape(NM), xs, gate_up_w, gate_up_w, down_w)

    out = pl.kernel(
        _sc_combine_body,
        out_type=jax.ShapeDtypeStruct((T, D), jnp.float32),
        mesh=mesh,
        scratch_types=[
            pltpu.VMEM((32, D), jnp.float32),
            pltpu.VMEM((32, D), jnp.float32),
            pltpu.VMEM((32,), jnp.int32),
            pltpu.VMEM((32, 16), jnp.float32),
            pltpu.VMEM((32, 16), jnp.float32),
            pltpu.SemaphoreType.DMA,
        ],
    )(ys, pos1, pos2, w1, w2)

    return out.reshape(B, S, D), aux.reshape(())


# speedup vs baseline: 1.6372x; 1.2154x over previous
import functools

import jax
import jax.numpy as jnp
from jax import lax
from jax.experimental import pallas as pl
from jax.experimental.pallas import tpu as pltpu
from jax.experimental.pallas import tpu_sc as plsc

TEMP = 1.0
LB_WEIGHT = 0.01

NC = 2
NS = 16
NW = NC * NS


def _routing_body(x_ref, gw_ref, pos1_ref, pos2_ref, w1_ref, w2_ref,
                  bs_ref, aux_ref, *, bm):
    E = gw_ref.shape[0]
    T = x_ref.shape[0]
    logits = lax.dot_general(
        x_ref[...], gw_ref[...],
        dimension_numbers=(((1,), (1,)), ((), ())),
        preferred_element_type=jnp.float32) / TEMP
    mx = jnp.max(logits, axis=1, keepdims=True)
    ex = jnp.exp(logits - mx)
    p = ex / jnp.sum(ex, axis=1, keepdims=True)
    eidx = lax.broadcasted_iota(jnp.int32, (T, E), 1)
    big = jnp.int32(E)
    p1 = jnp.max(p, axis=1, keepdims=True)
    i1 = jnp.min(jnp.where(p == p1, eidx, big), axis=1, keepdims=True)
    mask1 = eidx == i1
    pm = jnp.where(mask1, -1.0, p)
    p2 = jnp.max(pm, axis=1, keepdims=True)
    i2 = jnp.min(jnp.where(pm == p2, eidx, big), axis=1, keepdims=True)
    mask2 = eidx == i2
    norm = p1 + p2 + 1e-9
    w1_ref[...] = jnp.broadcast_to(p1 / norm, (T, 16))
    w2_ref[...] = jnp.broadcast_to(p2 / norm, (T, 16))

    oh = (mask1 | mask2).astype(jnp.float32)
    frac = jnp.sum(oh, axis=0, keepdims=True) / (T * 2)
    meanp = jnp.mean(p, axis=0, keepdims=True)
    aux_ref[...] = jnp.sum(frac * meanp).reshape(1, 1) * (LB_WEIGHT * E)

    CH = 512
    rr = lax.broadcasted_iota(jnp.int32, (CH, CH), 0)
    cc = lax.broadcasted_iota(jnp.int32, (CH, CH), 1)
    tril = (cc < rr).astype(jnp.bfloat16)
    pieces = []
    carry = jnp.zeros((1, E), jnp.float32)
    for c in range(T // CH):
        ohc = oh[c * CH:(c + 1) * CH, :]
        local = lax.dot_general(
            tril, ohc.astype(jnp.bfloat16),
            dimension_numbers=(((1,), (0,)), ((), ())),
            preferred_element_type=jnp.float32)
        pieces.append(local + carry)
        carry = carry + jnp.sum(ohc, axis=0, keepdims=True)
    cnt_before = jnp.concatenate(pieces, axis=0)
    counts = carry
    pc = jnp.ceil(counts / bm) * bm
    trE = (lax.broadcasted_iota(jnp.int32, (E, E), 0)
           < lax.broadcasted_iota(jnp.int32, (E, E), 1)).astype(jnp.float32)
    pad_off = lax.dot_general(
        pc, trE, dimension_numbers=(((1,), (0,)), ((), ())),
        preferred_element_type=jnp.float32)
    base1 = jnp.sum(jnp.where(mask1, pad_off, 0.0), axis=1, keepdims=True)
    base2 = jnp.sum(jnp.where(mask2, pad_off, 0.0), axis=1, keepdims=True)
    rank1 = jnp.sum(jnp.where(mask1, cnt_before, 0.0), axis=1, keepdims=True)
    rank2 = jnp.sum(jnp.where(mask2, cnt_before, 0.0), axis=1, keepdims=True)
    pos1_ref[...] = (base1 + rank1).astype(jnp.int32)
    pos2_ref[...] = (base2 + rank2).astype(jnp.int32)

    total = pad_off[:, E - 1:E] + pc[:, E - 1:E]
    bs_ref[...] = (jnp.concatenate([pad_off, total], axis=1) / bm
                   ).astype(jnp.int32)


def _sc_scatter_body(x_hbm, pos1_hbm, pos2_hbm, xs_hbm, rows_v, idx_v, sem):
    wid = lax.axis_index("s") * NC + lax.axis_index("c")
    T = x_hbm.shape[0]
    tpw = T // NW
    base = wid * tpw
    pltpu.sync_copy(x_hbm.at[pl.ds(base, tpw)], rows_v)
    pltpu.sync_copy(pos1_hbm.at[pl.ds(base, tpw)], idx_v)
    pltpu.async_copy(rows_v, xs_hbm.at[idx_v], sem).wait()
    pltpu.sync_copy(pos2_hbm.at[pl.ds(base, tpw)], idx_v)
    pltpu.async_copy(rows_v, xs_hbm.at[idx_v], sem).wait()


def _gemm_body(bs_ref, xs_hbm, wgu_hbm, wd_hbm, out_hbm,
               xs_v, out_v, stage_b, wg_b, wu_b, wd_b, wsem, xsem, osem,
               *, bm, bn, nj):
    j = pl.program_id(0)
    E = wgu_hbm.shape[0]
    D = wd_hbm.shape[1]
    DFF = wd_hbm.shape[2]

    def w_copies(jj, e, slot):
        return (
            pltpu.make_async_copy(
                wgu_hbm.at[e, pl.ds(jj * bn, bn), :], wg_b.at[slot],
                wsem.at[slot]),
            pltpu.make_async_copy(
                wgu_hbm.at[e, pl.ds(DFF + jj * bn, bn), :], wu_b.at[slot],
                wsem.at[slot]),
            pltpu.make_async_copy(
                wd_hbm.at[e, :, pl.ds(jj * bn, bn)], wd_b.at[slot],
                wsem.at[slot]),
        )

    MPAD = xs_hbm.shape[0]
    CHR = 512

    @pl.when(j == 0)
    def _prologue():
        for c in w_copies(j, 0, 0):
            c.start()
        nch = MPAD // CHR

        def stage_copy(c):
            return pltpu.make_async_copy(
                xs_hbm.at[pl.ds(c * CHR, CHR), :], stage_b.at[c % 2], xsem)

        for c in range(nch):
            cp = stage_copy(c)
            cp.start()
            cp.wait()
            xs_v[pl.ds(c * CHR, CHR), :] = (
                stage_b[c % 2].astype(jnp.bfloat16))

    for e in range(E):
        if e + 1 < E:
            for c in w_copies(j, e + 1, (e + 1) % 2):
                c.start()
        else:
            @pl.when(j + 1 < nj)
            def _prefetch_next_j():
                for c in w_copies(j + 1, 0, 0):
                    c.start()
        for c in w_copies(j, e, e % 2):
            c.wait()
        wg = wg_b[e % 2].astype(jnp.bfloat16)
        wu = wu_b[e % 2].astype(jnp.bfloat16)
        wd = wd_b[e % 2].astype(jnp.bfloat16)

        def blk(i, _):
            xb = xs_v[pl.ds(i * bm, bm), :]
            g = lax.dot_general(xb, wg, (((1,), (1,)), ((), ())),
                                preferred_element_type=jnp.float32)
            u = lax.dot_general(xb, wu, (((1,), (1,)), ((), ())),
                                preferred_element_type=jnp.float32)
            h = (g * lax.logistic(g) * u).astype(jnp.bfloat16)
            y = lax.dot_general(h, wd, (((1,), (1,)), ((), ())),
                                preferred_element_type=jnp.float32)

            @pl.when(j == 0)
            def _store():
                out_v[pl.ds(i * bm, bm), :] = y

            @pl.when(j > 0)
            def _accum():
                out_v[pl.ds(i * bm, bm), :] += y

            return 0

        lax.fori_loop(bs_ref[0, e], bs_ref[0, e + 1], blk, 0)

    @pl.when(j == nj - 1)
    def _epilogue():
        cp = pltpu.make_async_copy(out_v, out_hbm, osem)
        cp.start()
        cp.wait()


def _sc_combine_body(ys_hbm, pos1_hbm, pos2_hbm, w1_hbm, w2_hbm, out_hbm,
                     a_v, b_v, idx_v, w1_v, w2_v, sem):
    wid = lax.axis_index("s") * NC + lax.axis_index("c")
    T = out_hbm.shape[0]
    D = out_hbm.shape[1]
    tpw = T // NW
    CHT = 32
    for ci in range(tpw // CHT):
        base = wid * tpw + ci * CHT
        pltpu.sync_copy(pos1_hbm.at[pl.ds(base, CHT)], idx_v)
        pltpu.async_copy(ys_hbm.at[idx_v], a_v, sem).wait()
        pltpu.sync_copy(pos2_hbm.at[pl.ds(base, CHT)], idx_v)
        pltpu.async_copy(ys_hbm.at[idx_v], b_v, sem).wait()
        pltpu.sync_copy(w1_hbm.at[pl.ds(base, CHT)], w1_v)
        pltpu.sync_copy(w2_hbm.at[pl.ds(base, CHT)], w2_v)

        def row_body(r, _):
            s1 = w1_v[r, :]
            s2 = w2_v[r, :]
            for cj in range(D // 16):
                sl = pl.ds(cj * 16, 16)
                a_v[r, sl] = a_v[r, sl] * s1 + b_v[r, sl] * s2
            return 0

        lax.fori_loop(0, CHT, row_body, 0)
        pltpu.sync_copy(a_v, out_hbm.at[pl.ds(base, CHT)])


def kernel(x, gate_w, gate_up_w, down_w):
    B, S, D = x.shape
    T = B * S
    E = gate_w.shape[0]
    DFF = down_w.shape[2]
    K = 2
    BM = 128
    BN = 512
    NM = (T * K) // BM + E
    MPAD = NM * BM
    J = DFF // BN
    JG = DFF // BN
    x_flat = x.reshape(T, D)

    pos1, pos2, w1, w2, bs, aux = pl.pallas_call(
        functools.partial(_routing_body, bm=BM),
        out_shape=(
            jax.ShapeDtypeStruct((T, 1), jnp.int32),
            jax.ShapeDtypeStruct((T, 1), jnp.int32),
            jax.ShapeDtypeStruct((T, 16), jnp.float32),
            jax.ShapeDtypeStruct((T, 16), jnp.float32),
            jax.ShapeDtypeStruct((1, E + 1), jnp.int32),
            jax.ShapeDtypeStruct((1, 1), jnp.float32),
        ),
    )(x_flat, gate_w)
    pos1 = pos1.reshape(T)
    pos2 = pos2.reshape(T)

    mesh = plsc.VectorSubcoreMesh(core_axis_name="c", subcore_axis_name="s")
    xs = pl.kernel(
        _sc_scatter_body,
        out_type=jax.ShapeDtypeStruct((MPAD, D), jnp.float32),
        mesh=mesh,
        scratch_types=[
            pltpu.VMEM((T // NW, D), jnp.float32),
            pltpu.VMEM((T // NW,), jnp.int32),
            pltpu.SemaphoreType.DMA,
        ],
    )(x_flat, pos1, pos2)

    ys = pl.pallas_call(
        functools.partial(_gemm_body, bm=BM, bn=BN, nj=J),
        grid=(J,),
        in_specs=[
            pl.BlockSpec(memory_space=pltpu.SMEM),
            pl.BlockSpec(memory_space=pl.ANY),
            pl.BlockSpec(memory_space=pl.ANY),
            pl.BlockSpec(memory_space=pl.ANY),
        ],
        out_specs=pl.BlockSpec(memory_space=pl.ANY),
        out_shape=jax.ShapeDtypeStruct((MPAD, D), jnp.float32),
        scratch_shapes=[
            pltpu.VMEM((MPAD, D), jnp.bfloat16),
            pltpu.VMEM((MPAD, D), jnp.float32),
            pltpu.VMEM((2, 512, D), jnp.float32),
            pltpu.VMEM((2, BN, D), jnp.float32),
            pltpu.VMEM((2, BN, D), jnp.float32),
            pltpu.VMEM((2, D, BN), jnp.float32),
            pltpu.SemaphoreType.DMA((2,)),
            pltpu.SemaphoreType.DMA,
            pltpu.SemaphoreType.DMA,
        ],
    )(bs, xs, gate_up_w, down_w)

    out = pl.kernel(
        _sc_combine_body,
        out_type=jax.ShapeDtypeStruct((T, D), jnp.float32),
        mesh=mesh,
        scratch_types=[
            pltpu.VMEM((32, D), jnp.float32),
            pltpu.VMEM((32, D), jnp.float32),
            pltpu.VMEM((32,), jnp.int32),
            pltpu.VMEM((32, 16), jnp.float32),
            pltpu.VMEM((32, 16), jnp.float32),
            pltpu.SemaphoreType.DMA,
        ],
    )(ys, pos1, pos2, w1, w2)

    return out.reshape(B, S, D), aux.reshape(())


# speedup vs baseline: 1.9870x; 1.2137x over previous
import functools

import jax
import jax.numpy as jnp
from jax import lax
from jax.experimental import pallas as pl
from jax.experimental.pallas import tpu as pltpu
from jax.experimental.pallas import tpu_sc as plsc

TEMP = 1.0
LB_WEIGHT = 0.01

NC = 2
NS = 16
NW = NC * NS


def _routing_body(x_ref, gw_ref, pos1_ref, pos2_ref, w1_ref, w2_ref,
                  bs_ref, aux_ref, *, bm):
    E = gw_ref.shape[0]
    T = x_ref.shape[0]
    logits = lax.dot_general(
        x_ref[...], gw_ref[...],
        dimension_numbers=(((1,), (1,)), ((), ())),
        preferred_element_type=jnp.float32) / TEMP
    mx = jnp.max(logits, axis=1, keepdims=True)
    ex = jnp.exp(logits - mx)
    p = ex / jnp.sum(ex, axis=1, keepdims=True)
    eidx = lax.broadcasted_iota(jnp.int32, (T, E), 1)
    big = jnp.int32(E)
    p1 = jnp.max(p, axis=1, keepdims=True)
    i1 = jnp.min(jnp.where(p == p1, eidx, big), axis=1, keepdims=True)
    mask1 = eidx == i1
    pm = jnp.where(mask1, -1.0, p)
    p2 = jnp.max(pm, axis=1, keepdims=True)
    i2 = jnp.min(jnp.where(pm == p2, eidx, big), axis=1, keepdims=True)
    mask2 = eidx == i2
    norm = p1 + p2 + 1e-9
    w1_ref[...] = jnp.broadcast_to(p1 / norm, (T, 16))
    w2_ref[...] = jnp.broadcast_to(p2 / norm, (T, 16))

    oh = (mask1 | mask2).astype(jnp.float32)
    frac = jnp.sum(oh, axis=0, keepdims=True) / (T * 2)
    meanp = jnp.mean(p, axis=0, keepdims=True)
    aux_ref[...] = jnp.sum(frac * meanp).reshape(1, 1) * (LB_WEIGHT * E)

    CH = 512
    rr = lax.broadcasted_iota(jnp.int32, (CH, CH), 0)
    cc = lax.broadcasted_iota(jnp.int32, (CH, CH), 1)
    tril = (cc < rr).astype(jnp.bfloat16)
    pieces = []
    carry = jnp.zeros((1, E), jnp.float32)
    for c in range(T // CH):
        ohc = oh[c * CH:(c + 1) * CH, :]
        local = lax.dot_general(
            tril, ohc.astype(jnp.bfloat16),
            dimension_numbers=(((1,), (0,)), ((), ())),
            preferred_element_type=jnp.float32)
        pieces.append(local + carry)
        carry = carry + jnp.sum(ohc, axis=0, keepdims=True)
    cnt_before = jnp.concatenate(pieces, axis=0)
    counts = carry
    pc = jnp.ceil(counts / bm) * bm
    trE = (lax.broadcasted_iota(jnp.int32, (E, E), 0)
           < lax.broadcasted_iota(jnp.int32, (E, E), 1)).astype(jnp.float32)
    pad_off = lax.dot_general(
        pc, trE, dimension_numbers=(((1,), (0,)), ((), ())),
        preferred_element_type=jnp.float32)
    base1 = jnp.sum(jnp.where(mask1, pad_off, 0.0), axis=1, keepdims=True)
    base2 = jnp.sum(jnp.where(mask2, pad_off, 0.0), axis=1, keepdims=True)
    rank1 = jnp.sum(jnp.where(mask1, cnt_before, 0.0), axis=1, keepdims=True)
    rank2 = jnp.sum(jnp.where(mask2, cnt_before, 0.0), axis=1, keepdims=True)
    pos1_ref[...] = (base1 + rank1).astype(jnp.int32)
    pos2_ref[...] = (base2 + rank2).astype(jnp.int32)

    total = pad_off[:, E - 1:E] + pc[:, E - 1:E]
    bs_ref[...] = (jnp.concatenate([pad_off, total], axis=1) / bm
                   ).astype(jnp.int32)


def _sc_scatter_body(x_hbm, pos1_hbm, pos2_hbm, xs_hbm, rows_v, idx_v, sem):
    wid = lax.axis_index("s") * NC + lax.axis_index("c")
    T = x_hbm.shape[0]
    tpw = T // NW
    base = wid * tpw
    pltpu.sync_copy(x_hbm.at[pl.ds(base, tpw)], rows_v)
    pltpu.sync_copy(pos1_hbm.at[pl.ds(base, tpw)], idx_v)
    pltpu.async_copy(rows_v, xs_hbm.at[idx_v], sem).wait()
    pltpu.sync_copy(pos2_hbm.at[pl.ds(base, tpw)], idx_v)
    pltpu.async_copy(rows_v, xs_hbm.at[idx_v], sem).wait()


def _gemm_body(bs_ref, xs_hbm, wgu_hbm, wd_hbm, out_hbm,
               xs_v, out_v, stage_b, wg_b, wu_b, wd_b, wsem, xsem, osem,
               *, bm, bn, nj):
    j = pl.program_id(0)
    E = wgu_hbm.shape[0]
    D = wd_hbm.shape[1]
    DFF = wd_hbm.shape[2]

    def w_copies(jj, e, slot):
        return (
            pltpu.make_async_copy(
                wgu_hbm.at[e, pl.ds(jj * bn, bn), :], wg_b.at[slot],
                wsem.at[slot]),
            pltpu.make_async_copy(
                wgu_hbm.at[e, pl.ds(DFF + jj * bn, bn), :], wu_b.at[slot],
                wsem.at[slot]),
            pltpu.make_async_copy(
                wd_hbm.at[e, :, pl.ds(jj * bn, bn)], wd_b.at[slot],
                wsem.at[slot]),
        )

    MPAD = xs_hbm.shape[0]
    CHR = 256

    @pl.when(j == 0)
    def _prologue():
        for c in w_copies(j, 0, 0):
            c.start()
        nch = MPAD // CHR

        def stage_copy(c):
            return pltpu.make_async_copy(
                xs_hbm.at[pl.ds(c * CHR, CHR), :], stage_b.at[0], xsem)

        for c in range(nch):
            cp = stage_copy(c)
            cp.start()
            cp.wait()
            xs_v[pl.ds(c * CHR, CHR), :] = (
                stage_b[0].astype(jnp.bfloat16))

    for e in range(E):
        if e + 1 < E:
            for c in w_copies(j, e + 1, (e + 1) % 2):
                c.start()
        else:
            @pl.when(j + 1 < nj)
            def _prefetch_next_j():
                for c in w_copies(j + 1, 0, 0):
                    c.start()
        for c in w_copies(j, e, e % 2):
            c.wait()
        wg = wg_b[e % 2].astype(jnp.bfloat16)
        wu = wu_b[e % 2].astype(jnp.bfloat16)
        wd = wd_b[e % 2].astype(jnp.bfloat16)

        def blk(i, _):
            xb = xs_v[pl.ds(i * bm, bm), :]
            g = lax.dot_general(xb, wg, (((1,), (1,)), ((), ())),
                                preferred_element_type=jnp.float32)
            u = lax.dot_general(xb, wu, (((1,), (1,)), ((), ())),
                                preferred_element_type=jnp.float32)
            h = (g * lax.logistic(g) * u).astype(jnp.bfloat16)
            y = lax.dot_general(h, wd, (((1,), (1,)), ((), ())),
                                preferred_element_type=jnp.float32)

            @pl.when(j == 0)
            def _store():
                out_v[pl.ds(i * bm, bm), :] = y

            @pl.when(j > 0)
            def _accum():
                out_v[pl.ds(i * bm, bm), :] += y

            return 0

        lax.fori_loop(bs_ref[0, e], bs_ref[0, e + 1], blk, 0)

    @pl.when(j == nj - 1)
    def _epilogue():
        cp = pltpu.make_async_copy(out_v, out_hbm, osem)
        cp.start()
        cp.wait()


def _sc_combine_body(ys_hbm, pos1_hbm, pos2_hbm, w1_hbm, w2_hbm, out_hbm,
                     a_v, b_v, idx_v, w1_v, w2_v, sem):
    wid = lax.axis_index("s") * NC + lax.axis_index("c")
    T = out_hbm.shape[0]
    D = out_hbm.shape[1]
    tpw = T // NW
    CHT = 32
    for ci in range(tpw // CHT):
        base = wid * tpw + ci * CHT
        pltpu.sync_copy(pos1_hbm.at[pl.ds(base, CHT)], idx_v)
        pltpu.async_copy(ys_hbm.at[idx_v], a_v, sem).wait()
        pltpu.sync_copy(pos2_hbm.at[pl.ds(base, CHT)], idx_v)
        pltpu.async_copy(ys_hbm.at[idx_v], b_v, sem).wait()
        pltpu.sync_copy(w1_hbm.at[pl.ds(base, CHT)], w1_v)
        pltpu.sync_copy(w2_hbm.at[pl.ds(base, CHT)], w2_v)

        def row_body(r, _):
            s1 = w1_v[r, :]
            s2 = w2_v[r, :]
            for cj in range(D // 16):
                sl = pl.ds(cj * 16, 16)
                a_v[r, sl] = a_v[r, sl] * s1 + b_v[r, sl] * s2
            return 0

        lax.fori_loop(0, CHT, row_body, 0)
        pltpu.sync_copy(a_v, out_hbm.at[pl.ds(base, CHT)])


def kernel(x, gate_w, gate_up_w, down_w):
    B, S, D = x.shape
    T = B * S
    E = gate_w.shape[0]
    DFF = down_w.shape[2]
    K = 2
    BM = 256
    BN = 256
    NM = (T * K) // BM + E
    MPAD = NM * BM
    J = DFF // BN
    JG = DFF // BN
    x_flat = x.reshape(T, D)

    pos1, pos2, w1, w2, bs, aux = pl.pallas_call(
        functools.partial(_routing_body, bm=BM),
        out_shape=(
            jax.ShapeDtypeStruct((T, 1), jnp.int32),
            jax.ShapeDtypeStruct((T, 1), jnp.int32),
            jax.ShapeDtypeStruct((T, 16), jnp.float32),
            jax.ShapeDtypeStruct((T, 16), jnp.float32),
            jax.ShapeDtypeStruct((1, E + 1), jnp.int32),
            jax.ShapeDtypeStruct((1, 1), jnp.float32),
        ),
    )(x_flat, gate_w)
    pos1 = pos1.reshape(T)
    pos2 = pos2.reshape(T)

    mesh = plsc.VectorSubcoreMesh(core_axis_name="c", subcore_axis_name="s")
    xs = pl.kernel(
        _sc_scatter_body,
        out_type=jax.ShapeDtypeStruct((MPAD, D), jnp.float32),
        mesh=mesh,
        scratch_types=[
            pltpu.VMEM((T // NW, D), jnp.float32),
            pltpu.VMEM((T // NW,), jnp.int32),
            pltpu.SemaphoreType.DMA,
        ],
    )(x_flat, pos1, pos2)

    ys = pl.pallas_call(
        functools.partial(_gemm_body, bm=BM, bn=BN, nj=J),
        grid=(J,),
        in_specs=[
            pl.BlockSpec(memory_space=pltpu.SMEM),
            pl.BlockSpec(memory_space=pl.ANY),
            pl.BlockSpec(memory_space=pl.ANY),
            pl.BlockSpec(memory_space=pl.ANY),
        ],
        out_specs=pl.BlockSpec(memory_space=pl.ANY),
        out_shape=jax.ShapeDtypeStruct((MPAD, D), jnp.float32),
        scratch_shapes=[
            pltpu.VMEM((MPAD, D), jnp.bfloat16),
            pltpu.VMEM((MPAD, D), jnp.float32),
            pltpu.VMEM((1, 256, D), jnp.float32),
            pltpu.VMEM((2, BN, D), jnp.float32),
            pltpu.VMEM((2, BN, D), jnp.float32),
            pltpu.VMEM((2, D, BN), jnp.float32),
            pltpu.SemaphoreType.DMA((2,)),
            pltpu.SemaphoreType.DMA,
            pltpu.SemaphoreType.DMA,
        ],
    )(bs, xs, gate_up_w, down_w)

    out = pl.kernel(
        _sc_combine_body,
        out_type=jax.ShapeDtypeStruct((T, D), jnp.float32),
        mesh=mesh,
        scratch_types=[
            pltpu.VMEM((32, D), jnp.float32),
            pltpu.VMEM((32, D), jnp.float32),
            pltpu.VMEM((32,), jnp.int32),
            pltpu.VMEM((32, 16), jnp.float32),
            pltpu.VMEM((32, 16), jnp.float32),
            pltpu.SemaphoreType.DMA,
        ],
    )(ys, pos1, pos2, w1, w2)

    return out.reshape(B, S, D), aux.reshape(())


# speedup vs baseline: 2.0183x; 1.0157x over previous
import functools

import jax
import jax.numpy as jnp
from jax import lax
from jax.experimental import pallas as pl
from jax.experimental.pallas import tpu as pltpu
from jax.experimental.pallas import tpu_sc as plsc

TEMP = 1.0
LB_WEIGHT = 0.01

NC = 2
NS = 16
NW = NC * NS


def _routing_body(x_ref, gw_ref, pos1_ref, pos2_ref, w1_ref, w2_ref,
                  bs_ref, aux_ref, *, bm):
    E = gw_ref.shape[0]
    T = x_ref.shape[0]
    logits = lax.dot_general(
        x_ref[...], gw_ref[...],
        dimension_numbers=(((1,), (1,)), ((), ())),
        preferred_element_type=jnp.float32) / TEMP
    mx = jnp.max(logits, axis=1, keepdims=True)
    ex = jnp.exp(logits - mx)
    p = ex / jnp.sum(ex, axis=1, keepdims=True)
    eidx = lax.broadcasted_iota(jnp.int32, (T, E), 1)
    big = jnp.int32(E)
    p1 = jnp.max(p, axis=1, keepdims=True)
    i1 = jnp.min(jnp.where(p == p1, eidx, big), axis=1, keepdims=True)
    mask1 = eidx == i1
    pm = jnp.where(mask1, -1.0, p)
    p2 = jnp.max(pm, axis=1, keepdims=True)
    i2 = jnp.min(jnp.where(pm == p2, eidx, big), axis=1, keepdims=True)
    mask2 = eidx == i2
    norm = p1 + p2 + 1e-9
    w1_ref[...] = jnp.broadcast_to(p1 / norm, (T, 16))
    w2_ref[...] = jnp.broadcast_to(p2 / norm, (T, 16))

    oh = (mask1 | mask2).astype(jnp.float32)
    frac = jnp.sum(oh, axis=0, keepdims=True) / (T * 2)
    meanp = jnp.mean(p, axis=0, keepdims=True)
    aux_ref[...] = jnp.sum(frac * meanp).reshape(1, 1) * (LB_WEIGHT * E)

    CH = 512
    rr = lax.broadcasted_iota(jnp.int32, (CH, CH), 0)
    cc = lax.broadcasted_iota(jnp.int32, (CH, CH), 1)
    tril = (cc < rr).astype(jnp.bfloat16)
    pieces = []
    carry = jnp.zeros((1, E), jnp.float32)
    for c in range(T // CH):
        ohc = oh[c * CH:(c + 1) * CH, :]
        local = lax.dot_general(
            tril, ohc.astype(jnp.bfloat16),
            dimension_numbers=(((1,), (0,)), ((), ())),
            preferred_element_type=jnp.float32)
        pieces.append(local + carry)
        carry = carry + jnp.sum(ohc, axis=0, keepdims=True)
    cnt_before = jnp.concatenate(pieces, axis=0)
    counts = carry
    pc = jnp.ceil(counts / bm) * bm
    trE = (lax.broadcasted_iota(jnp.int32, (E, E), 0)
           < lax.broadcasted_iota(jnp.int32, (E, E), 1)).astype(jnp.float32)
    pad_off = lax.dot_general(
        pc, trE, dimension_numbers=(((1,), (0,)), ((), ())),
        preferred_element_type=jnp.float32)
    base1 = jnp.sum(jnp.where(mask1, pad_off, 0.0), axis=1, keepdims=True)
    base2 = jnp.sum(jnp.where(mask2, pad_off, 0.0), axis=1, keepdims=True)
    rank1 = jnp.sum(jnp.where(mask1, cnt_before, 0.0), axis=1, keepdims=True)
    rank2 = jnp.sum(jnp.where(mask2, cnt_before, 0.0), axis=1, keepdims=True)
    pos1_ref[...] = (base1 + rank1).astype(jnp.int32)
    pos2_ref[...] = (base2 + rank2).astype(jnp.int32)

    total = pad_off[:, E - 1:E] + pc[:, E - 1:E]
    bs_ref[...] = (jnp.concatenate([pad_off, total], axis=1) / bm
                   ).astype(jnp.int32)


def _sc_scatter_body(x_hbm, pos1_hbm, pos2_hbm, xs_hbm, rows_v, idx_v, sem):
    wid = lax.axis_index("s") * NC + lax.axis_index("c")
    T = x_hbm.shape[0]
    tpw = T // NW
    base = wid * tpw
    pltpu.sync_copy(x_hbm.at[pl.ds(base, tpw)], rows_v)
    pltpu.sync_copy(pos1_hbm.at[pl.ds(base, tpw)], idx_v)
    pltpu.async_copy(rows_v, xs_hbm.at[idx_v], sem).wait()
    pltpu.sync_copy(pos2_hbm.at[pl.ds(base, tpw)], idx_v)
    pltpu.async_copy(rows_v, xs_hbm.at[idx_v], sem).wait()


def _gemm_body(bs_ref, xs_hbm, wgu_hbm, wd_hbm, out_hbm,
               xs_v, out_v, stage_b, wg_b, wu_b, wd_b,
               wgbf_v, wubf_v, wdbf_v, hbf_v, wsem, xsem, osem,
               *, bm, bn, nj):
    j = pl.program_id(0)
    E = wgu_hbm.shape[0]
    D = wd_hbm.shape[1]
    DFF = wd_hbm.shape[2]

    def w_copies(jj, e, slot):
        return (
            pltpu.make_async_copy(
                wgu_hbm.at[e, pl.ds(jj * bn, bn), :], wg_b.at[slot],
                wsem.at[slot]),
            pltpu.make_async_copy(
                wgu_hbm.at[e, pl.ds(DFF + jj * bn, bn), :], wu_b.at[slot],
                wsem.at[slot]),
            pltpu.make_async_copy(
                wd_hbm.at[e, :, pl.ds(jj * bn, bn)], wd_b.at[slot],
                wsem.at[slot]),
        )

    MPAD = xs_hbm.shape[0]
    CHR = 256

    @pl.when(j == 0)
    def _prologue():
        for c in w_copies(j, 0, 0):
            c.start()
        nch = MPAD // CHR

        def stage_copy(c):
            return pltpu.make_async_copy(
                xs_hbm.at[pl.ds(c * CHR, CHR), :], stage_b.at[0], xsem)

        for c in range(nch):
            cp = stage_copy(c)
            cp.start()
            cp.wait()
            xs_v[pl.ds(c * CHR, CHR), :] = (
                stage_b[0].astype(jnp.bfloat16))

    for e in range(E):
        if e + 1 < E:
            for c in w_copies(j, e + 1, (e + 1) % 2):
                c.start()
        else:
            @pl.when(j + 1 < nj)
            def _prefetch_next_j():
                for c in w_copies(j + 1, 0, 0):
                    c.start()
        for c in w_copies(j, e, e % 2):
            c.wait()
        wgbf_v[...] = wg_b[e % 2].astype(jnp.bfloat16)
        wubf_v[...] = wu_b[e % 2].astype(jnp.bfloat16)
        wdbf_v[...] = wd_b[e % 2].astype(jnp.bfloat16)

        def blk(i, _):
            xb = xs_v[pl.ds(i * bm, bm), :]
            g = lax.dot_general(xb, wgbf_v[...], (((1,), (1,)), ((), ())),
                                preferred_element_type=jnp.float32)
            u = lax.dot_general(xb, wubf_v[...], (((1,), (1,)), ((), ())),
                                preferred_element_type=jnp.float32)
            hbf_v[...] = (g * lax.logistic(g) * u).astype(jnp.bfloat16)
            y = lax.dot_general(hbf_v[...], wdbf_v[...],
                                (((1,), (1,)), ((), ())),
                                preferred_element_type=jnp.float32)

            @pl.when(j == 0)
            def _store():
                out_v[pl.ds(i * bm, bm), :] = y

            @pl.when(j > 0)
            def _accum():
                out_v[pl.ds(i * bm, bm), :] += y

            return 0

        lax.fori_loop(bs_ref[0, e], bs_ref[0, e + 1], blk, 0)

    @pl.when(j == nj - 1)
    def _epilogue():
        cp = pltpu.make_async_copy(out_v, out_hbm, osem)
        cp.start()
        cp.wait()


def _sc_combine_body(ys_hbm, pos1_hbm, pos2_hbm, w1_hbm, w2_hbm, out_hbm,
                     a_v, b_v, idx_v, w1_v, w2_v, sem):
    wid = lax.axis_index("s") * NC + lax.axis_index("c")
    T = out_hbm.shape[0]
    D = out_hbm.shape[1]
    tpw = T // NW
    CHT = 32
    for ci in range(tpw // CHT):
        base = wid * tpw + ci * CHT
        pltpu.sync_copy(pos1_hbm.at[pl.ds(base, CHT)], idx_v)
        pltpu.async_copy(ys_hbm.at[idx_v], a_v, sem).wait()
        pltpu.sync_copy(pos2_hbm.at[pl.ds(base, CHT)], idx_v)
        pltpu.async_copy(ys_hbm.at[idx_v], b_v, sem).wait()
        pltpu.sync_copy(w1_hbm.at[pl.ds(base, CHT)], w1_v)
        pltpu.sync_copy(w2_hbm.at[pl.ds(base, CHT)], w2_v)

        def row_body(r, _):
            s1 = w1_v[r, :]
            s2 = w2_v[r, :]
            for cj in range(D // 16):
                sl = pl.ds(cj * 16, 16)
                a_v[r, sl] = a_v[r, sl] * s1 + b_v[r, sl] * s2
            return 0

        lax.fori_loop(0, CHT, row_body, 0)
        pltpu.sync_copy(a_v, out_hbm.at[pl.ds(base, CHT)])


def kernel(x, gate_w, gate_up_w, down_w):
    B, S, D = x.shape
    T = B * S
    E = gate_w.shape[0]
    DFF = down_w.shape[2]
    K = 2
    BM = 256
    BN = 256
    NM = (T * K) // BM + E
    MPAD = NM * BM
    J = DFF // BN
    JG = DFF // BN
    x_flat = x.reshape(T, D)

    pos1, pos2, w1, w2, bs, aux = pl.pallas_call(
        functools.partial(_routing_body, bm=BM),
        out_shape=(
            jax.ShapeDtypeStruct((T, 1), jnp.int32),
            jax.ShapeDtypeStruct((T, 1), jnp.int32),
            jax.ShapeDtypeStruct((T, 16), jnp.float32),
            jax.ShapeDtypeStruct((T, 16), jnp.float32),
            jax.ShapeDtypeStruct((1, E + 1), jnp.int32),
            jax.ShapeDtypeStruct((1, 1), jnp.float32),
        ),
    )(x_flat, gate_w)
    pos1 = pos1.reshape(T)
    pos2 = pos2.reshape(T)

    mesh = plsc.VectorSubcoreMesh(core_axis_name="c", subcore_axis_name="s")
    xs = pl.kernel(
        _sc_scatter_body,
        out_type=jax.ShapeDtypeStruct((MPAD, D), jnp.float32),
        mesh=mesh,
        scratch_types=[
            pltpu.VMEM((T // NW, D), jnp.float32),
            pltpu.VMEM((T // NW,), jnp.int32),
            pltpu.SemaphoreType.DMA,
        ],
    )(x_flat, pos1, pos2)

    ys = pl.pallas_call(
        functools.partial(_gemm_body, bm=BM, bn=BN, nj=J),
        grid=(J,),
        in_specs=[
            pl.BlockSpec(memory_space=pltpu.SMEM),
            pl.BlockSpec(memory_space=pl.ANY),
            pl.BlockSpec(memory_space=pl.ANY),
            pl.BlockSpec(memory_space=pl.ANY),
        ],
        out_specs=pl.BlockSpec(memory_space=pl.ANY),
        out_shape=jax.ShapeDtypeStruct((MPAD, D), jnp.float32),
        scratch_shapes=[
            pltpu.VMEM((MPAD, D), jnp.bfloat16),
            pltpu.VMEM((MPAD, D), jnp.float32),
            pltpu.VMEM((1, 256, D), jnp.float32),
            pltpu.VMEM((2, BN, D), jnp.float32),
            pltpu.VMEM((2, BN, D), jnp.float32),
            pltpu.VMEM((2, D, BN), jnp.float32),
            pltpu.VMEM((BN, D), jnp.bfloat16),
            pltpu.VMEM((BN, D), jnp.bfloat16),
            pltpu.VMEM((D, BN), jnp.bfloat16),
            pltpu.VMEM((BM, BN), jnp.bfloat16),
            pltpu.SemaphoreType.DMA((2,)),
            pltpu.SemaphoreType.DMA,
            pltpu.SemaphoreType.DMA,
        ],
    )(bs, xs, gate_up_w, down_w)

    out = pl.kernel(
        _sc_combine_body,
        out_type=jax.ShapeDtypeStruct((T, D), jnp.float32),
        mesh=mesh,
        scratch_types=[
            pltpu.VMEM((32, D), jnp.float32),
            pltpu.VMEM((32, D), jnp.float32),
            pltpu.VMEM((32,), jnp.int32),
            pltpu.VMEM((32, 16), jnp.float32),
            pltpu.VMEM((32, 16), jnp.float32),
            pltpu.SemaphoreType.DMA,
        ],
    )(ys, pos1, pos2, w1, w2)

    return out.reshape(B, S, D), aux.reshape(())


# speedup vs baseline: 2.4963x; 1.2369x over previous
import functools

import jax
import jax.numpy as jnp
from jax import lax
from jax.experimental import pallas as pl
from jax.experimental.pallas import tpu as pltpu
from jax.experimental.pallas import tpu_sc as plsc

TEMP = 1.0
LB_WEIGHT = 0.01

NC = 2
NS = 16
NW = NC * NS


def _routing_body(x_ref, gw_ref, pos1_ref, pos2_ref, w1_ref, w2_ref,
                  bs_ref, aux_ref, *, bm):
    E = gw_ref.shape[0]
    T = x_ref.shape[0]
    logits = lax.dot_general(
        x_ref[...], gw_ref[...],
        dimension_numbers=(((1,), (1,)), ((), ())),
        preferred_element_type=jnp.float32) / TEMP
    mx = jnp.max(logits, axis=1, keepdims=True)
    ex = jnp.exp(logits - mx)
    p = ex / jnp.sum(ex, axis=1, keepdims=True)
    eidx = lax.broadcasted_iota(jnp.int32, (T, E), 1)
    big = jnp.int32(E)
    p1 = jnp.max(p, axis=1, keepdims=True)
    i1 = jnp.min(jnp.where(p == p1, eidx, big), axis=1, keepdims=True)
    mask1 = eidx == i1
    pm = jnp.where(mask1, -1.0, p)
    p2 = jnp.max(pm, axis=1, keepdims=True)
    i2 = jnp.min(jnp.where(pm == p2, eidx, big), axis=1, keepdims=True)
    mask2 = eidx == i2
    norm = p1 + p2 + 1e-9
    w1_ref[...] = jnp.broadcast_to(p1 / norm, (T, 16))
    w2_ref[...] = jnp.broadcast_to(p2 / norm, (T, 16))

    oh = (mask1 | mask2).astype(jnp.float32)
    frac = jnp.sum(oh, axis=0, keepdims=True) / (T * 2)
    meanp = jnp.mean(p, axis=0, keepdims=True)
    aux_ref[...] = jnp.sum(frac * meanp).reshape(1, 1) * (LB_WEIGHT * E)

    CH = 512
    rr = lax.broadcasted_iota(jnp.int32, (CH, CH), 0)
    cc = lax.broadcasted_iota(jnp.int32, (CH, CH), 1)
    tril = (cc < rr).astype(jnp.bfloat16)
    pieces = []
    carry = jnp.zeros((1, E), jnp.float32)
    for c in range(T // CH):
        ohc = oh[c * CH:(c + 1) * CH, :]
        local = lax.dot_general(
            tril, ohc.astype(jnp.bfloat16),
            dimension_numbers=(((1,), (0,)), ((), ())),
            preferred_element_type=jnp.float32)
        pieces.append(local + carry)
        carry = carry + jnp.sum(ohc, axis=0, keepdims=True)
    cnt_before = jnp.concatenate(pieces, axis=0)
    counts = carry
    pc = jnp.ceil(counts / bm) * bm
    trE = (lax.broadcasted_iota(jnp.int32, (E, E), 0)
           < lax.broadcasted_iota(jnp.int32, (E, E), 1)).astype(jnp.float32)
    pad_off = lax.dot_general(
        pc, trE, dimension_numbers=(((1,), (0,)), ((), ())),
        preferred_element_type=jnp.float32)
    base1 = jnp.sum(jnp.where(mask1, pad_off, 0.0), axis=1, keepdims=True)
    base2 = jnp.sum(jnp.where(mask2, pad_off, 0.0), axis=1, keepdims=True)
    rank1 = jnp.sum(jnp.where(mask1, cnt_before, 0.0), axis=1, keepdims=True)
    rank2 = jnp.sum(jnp.where(mask2, cnt_before, 0.0), axis=1, keepdims=True)
    pos1_ref[...] = (base1 + rank1).astype(jnp.int32)
    pos2_ref[...] = (base2 + rank2).astype(jnp.int32)

    total = pad_off[:, E - 1:E] + pc[:, E - 1:E]
    bs_ref[...] = (jnp.concatenate([pad_off, total], axis=1) / bm
                   ).astype(jnp.int32)


def _sc_scatter_body(x_hbm, pos1_hbm, pos2_hbm, xs_hbm, rows_v, idx_v, sem):
    wid = lax.axis_index("s") * NC + lax.axis_index("c")
    T = x_hbm.shape[0]
    tpw = T // NW
    base = wid * tpw
    pltpu.sync_copy(x_hbm.at[pl.ds(base, tpw)], rows_v)
    pltpu.sync_copy(pos1_hbm.at[pl.ds(base, tpw)], idx_v)
    pltpu.async_copy(rows_v, xs_hbm.at[idx_v], sem).wait()
    pltpu.sync_copy(pos2_hbm.at[pl.ds(base, tpw)], idx_v)
    pltpu.async_copy(rows_v, xs_hbm.at[idx_v], sem).wait()


def _gemm_body(bs_ref, xs_hbm, wgu_hbm, wd_hbm, out_hbm,
               xs_v, out_v, stage_b, wg_b, wu_b, wd_b, wsem, xsem, osem,
               *, bm, bn, nj):
    j = pl.program_id(0)
    E = wgu_hbm.shape[0]
    D = wd_hbm.shape[1]
    DFF = wd_hbm.shape[2]

    def w_copies(jj, e, slot):
        return (
            pltpu.make_async_copy(
                wgu_hbm.at[e, pl.ds(jj * bn, bn), :], wg_b.at[slot],
                wsem.at[slot]),
            pltpu.make_async_copy(
                wgu_hbm.at[e, pl.ds(DFF + jj * bn, bn), :], wu_b.at[slot],
                wsem.at[slot]),
            pltpu.make_async_copy(
                wd_hbm.at[e, :, pl.ds(jj * bn, bn)], wd_b.at[slot],
                wsem.at[slot]),
        )

    MPAD = xs_hbm.shape[0]
    CHR = 256

    @pl.when(j == 0)
    def _prologue():
        for c in w_copies(j, 0, 0):
            c.start()
        nch = MPAD // CHR

        def stage_copy(c):
            return pltpu.make_async_copy(
                xs_hbm.at[pl.ds(c * CHR, CHR), :], stage_b.at[0], xsem)

        for c in range(nch):
            cp = stage_copy(c)
            cp.start()
            cp.wait()
            xs_v[pl.ds(c * CHR, CHR), :] = (
                stage_b[0].astype(jnp.bfloat16))

    for e in range(E):
        if e + 1 < E:
            for c in w_copies(j, e + 1, (e + 1) % 2):
                c.start()
        else:
            @pl.when(j + 1 < nj)
            def _prefetch_next_j():
                for c in w_copies(j + 1, 0, 0):
                    c.start()
        for c in w_copies(j, e, e % 2):
            c.wait()
        wg = wg_b[e % 2]
        wu = wu_b[e % 2]
        wd = wd_b[e % 2]

        def blk(i, _):
            xb = xs_v[pl.ds(i * bm, bm), :].astype(jnp.float32)
            g = lax.dot_general(xb, wg, (((1,), (1,)), ((), ())),
                                preferred_element_type=jnp.float32)
            u = lax.dot_general(xb, wu, (((1,), (1,)), ((), ())),
                                preferred_element_type=jnp.float32)
            h = g * lax.logistic(g) * u
            y = lax.dot_general(h, wd, (((1,), (1,)), ((), ())),
                                preferred_element_type=jnp.float32)

            @pl.when(j == 0)
            def _store():
                out_v[pl.ds(i * bm, bm), :] = y

            @pl.when(j > 0)
            def _accum():
                out_v[pl.ds(i * bm, bm), :] += y

            return 0

        lax.fori_loop(bs_ref[0, e], bs_ref[0, e + 1], blk, 0)

    @pl.when(j == nj - 1)
    def _epilogue():
        cp = pltpu.make_async_copy(out_v, out_hbm, osem)
        cp.start()
        cp.wait()


def _sc_combine_body(ys_hbm, pos1_hbm, pos2_hbm, w1_hbm, w2_hbm, out_hbm,
                     a_v, b_v, idx_v, w1_v, w2_v, sem):
    wid = lax.axis_index("s") * NC + lax.axis_index("c")
    T = out_hbm.shape[0]
    D = out_hbm.shape[1]
    tpw = T // NW
    CHT = 32
    for ci in range(tpw // CHT):
        base = wid * tpw + ci * CHT
        pltpu.sync_copy(pos1_hbm.at[pl.ds(base, CHT)], idx_v)
        pltpu.async_copy(ys_hbm.at[idx_v], a_v, sem).wait()
        pltpu.sync_copy(pos2_hbm.at[pl.ds(base, CHT)], idx_v)
        pltpu.async_copy(ys_hbm.at[idx_v], b_v, sem).wait()
        pltpu.sync_copy(w1_hbm.at[pl.ds(base, CHT)], w1_v)
        pltpu.sync_copy(w2_hbm.at[pl.ds(base, CHT)], w2_v)

        def row_body(r, _):
            s1 = w1_v[r, :]
            s2 = w2_v[r, :]
            for cj in range(D // 16):
                sl = pl.ds(cj * 16, 16)
                a_v[r, sl] = a_v[r, sl] * s1 + b_v[r, sl] * s2
            return 0

        lax.fori_loop(0, CHT, row_body, 0)
        pltpu.sync_copy(a_v, out_hbm.at[pl.ds(base, CHT)])


def kernel(x, gate_w, gate_up_w, down_w):
    B, S, D = x.shape
    T = B * S
    E = gate_w.shape[0]
    DFF = down_w.shape[2]
    K = 2
    BM = 256
    BN = 512
    NM = (T * K) // BM + E
    MPAD = NM * BM
    J = DFF // BN
    JG = DFF // BN
    x_flat = x.reshape(T, D)

    pos1, pos2, w1, w2, bs, aux = pl.pallas_call(
        functools.partial(_routing_body, bm=BM),
        out_shape=(
            jax.ShapeDtypeStruct((T, 1), jnp.int32),
            jax.ShapeDtypeStruct((T, 1), jnp.int32),
            jax.ShapeDtypeStruct((T, 16), jnp.float32),
            jax.ShapeDtypeStruct((T, 16), jnp.float32),
            jax.ShapeDtypeStruct((1, E + 1), jnp.int32),
            jax.ShapeDtypeStruct((1, 1), jnp.float32),
        ),
    )(x_flat, gate_w)
    pos1 = pos1.reshape(T)
    pos2 = pos2.reshape(T)

    mesh = plsc.VectorSubcoreMesh(core_axis_name="c", subcore_axis_name="s")
    xs = pl.kernel(
        _sc_scatter_body,
        out_type=jax.ShapeDtypeStruct((MPAD, D), jnp.float32),
        mesh=mesh,
        scratch_types=[
            pltpu.VMEM((T // NW, D), jnp.float32),
            pltpu.VMEM((T // NW,), jnp.int32),
            pltpu.SemaphoreType.DMA,
        ],
    )(x_flat, pos1, pos2)

    ys = pl.pallas_call(
        functools.partial(_gemm_body, bm=BM, bn=BN, nj=J),
        grid=(J,),
        in_specs=[
            pl.BlockSpec(memory_space=pltpu.SMEM),
            pl.BlockSpec(memory_space=pl.ANY),
            pl.BlockSpec(memory_space=pl.ANY),
            pl.BlockSpec(memory_space=pl.ANY),
        ],
        out_specs=pl.BlockSpec(memory_space=pl.ANY),
        out_shape=jax.ShapeDtypeStruct((MPAD, D), jnp.float32),
        scratch_shapes=[
            pltpu.VMEM((MPAD, D), jnp.bfloat16),
            pltpu.VMEM((MPAD, D), jnp.float32),
            pltpu.VMEM((1, 256, D), jnp.float32),
            pltpu.VMEM((2, BN, D), jnp.float32),
            pltpu.VMEM((2, BN, D), jnp.float32),
            pltpu.VMEM((2, D, BN), jnp.float32),
            pltpu.SemaphoreType.DMA((2,)),
            pltpu.SemaphoreType.DMA,
            pltpu.SemaphoreType.DMA,
        ],
    )(bs, xs, gate_up_w, down_w)

    out = pl.kernel(
        _sc_combine_body,
        out_type=jax.ShapeDtypeStruct((T, D), jnp.float32),
        mesh=mesh,
        scratch_types=[
            pltpu.VMEM((32, D), jnp.float32),
            pltpu.VMEM((32, D), jnp.float32),
            pltpu.VMEM((32,), jnp.int32),
            pltpu.VMEM((32, 16), jnp.float32),
            pltpu.VMEM((32, 16), jnp.float32),
            pltpu.SemaphoreType.DMA,
        ],
    )(ys, pos1, pos2, w1, w2)

    return out.reshape(B, S, D), aux.reshape(())


# speedup vs baseline: 2.8361x; 1.1361x over previous
import functools

import jax
import jax.numpy as jnp
from jax import lax
from jax.experimental import pallas as pl
from jax.experimental.pallas import tpu as pltpu
from jax.experimental.pallas import tpu_sc as plsc

TEMP = 1.0
LB_WEIGHT = 0.01

NC = 2
NS = 16
NW = NC * NS


def _routing_body(x_ref, gw_ref, pos1_ref, pos2_ref, w1_ref, w2_ref,
                  bs_ref, aux_ref, *, bm):
    E = gw_ref.shape[0]
    T = x_ref.shape[0]
    logits = lax.dot_general(
        x_ref[...], gw_ref[...],
        dimension_numbers=(((1,), (1,)), ((), ())),
        preferred_element_type=jnp.float32) / TEMP
    mx = jnp.max(logits, axis=1, keepdims=True)
    ex = jnp.exp(logits - mx)
    p = ex / jnp.sum(ex, axis=1, keepdims=True)
    eidx = lax.broadcasted_iota(jnp.int32, (T, E), 1)
    big = jnp.int32(E)
    p1 = jnp.max(p, axis=1, keepdims=True)
    i1 = jnp.min(jnp.where(p == p1, eidx, big), axis=1, keepdims=True)
    mask1 = eidx == i1
    pm = jnp.where(mask1, -1.0, p)
    p2 = jnp.max(pm, axis=1, keepdims=True)
    i2 = jnp.min(jnp.where(pm == p2, eidx, big), axis=1, keepdims=True)
    mask2 = eidx == i2
    norm = p1 + p2 + 1e-9
    w1_ref[...] = jnp.broadcast_to(p1 / norm, (T, 16))
    w2_ref[...] = jnp.broadcast_to(p2 / norm, (T, 16))

    oh = (mask1 | mask2).astype(jnp.float32)
    frac = jnp.sum(oh, axis=0, keepdims=True) / (T * 2)
    meanp = jnp.mean(p, axis=0, keepdims=True)
    aux_ref[...] = jnp.sum(frac * meanp).reshape(1, 1) * (LB_WEIGHT * E)

    CH = 512
    rr = lax.broadcasted_iota(jnp.int32, (CH, CH), 0)
    cc = lax.broadcasted_iota(jnp.int32, (CH, CH), 1)
    tril = (cc < rr).astype(jnp.bfloat16)
    pieces = []
    carry = jnp.zeros((1, E), jnp.float32)
    for c in range(T // CH):
        ohc = oh[c * CH:(c + 1) * CH, :]
        local = lax.dot_general(
            tril, ohc.astype(jnp.bfloat16),
            dimension_numbers=(((1,), (0,)), ((), ())),
            preferred_element_type=jnp.float32)
        pieces.append(local + carry)
        carry = carry + jnp.sum(ohc, axis=0, keepdims=True)
    cnt_before = jnp.concatenate(pieces, axis=0)
    counts = carry
    pc = jnp.ceil(counts / bm) * bm
    trE = (lax.broadcasted_iota(jnp.int32, (E, E), 0)
           < lax.broadcasted_iota(jnp.int32, (E, E), 1)).astype(jnp.float32)
    pad_off = lax.dot_general(
        pc, trE, dimension_numbers=(((1,), (0,)), ((), ())),
        preferred_element_type=jnp.float32)
    base1 = jnp.sum(jnp.where(mask1, pad_off, 0.0), axis=1, keepdims=True)
    base2 = jnp.sum(jnp.where(mask2, pad_off, 0.0), axis=1, keepdims=True)
    rank1 = jnp.sum(jnp.where(mask1, cnt_before, 0.0), axis=1, keepdims=True)
    rank2 = jnp.sum(jnp.where(mask2, cnt_before, 0.0), axis=1, keepdims=True)
    pos1_ref[...] = (base1 + rank1).astype(jnp.int32)
    pos2_ref[...] = (base2 + rank2).astype(jnp.int32)

    total = pad_off[:, E - 1:E] + pc[:, E - 1:E]
    bs_ref[...] = (jnp.concatenate([pad_off, total], axis=1) / bm
                   ).astype(jnp.int32)


def _sc_scatter_body(x_hbm, pos1_hbm, pos2_hbm, xs_hbm, rows_v, idx_v, sem):
    wid = lax.axis_index("s") * NC + lax.axis_index("c")
    T = x_hbm.shape[0]
    tpw = T // NW
    base = wid * tpw
    pltpu.sync_copy(x_hbm.at[pl.ds(base, tpw)], rows_v)
    pltpu.sync_copy(pos1_hbm.at[pl.ds(base, tpw)], idx_v)
    pltpu.async_copy(rows_v, xs_hbm.at[idx_v], sem).wait()
    pltpu.sync_copy(pos2_hbm.at[pl.ds(base, tpw)], idx_v)
    pltpu.async_copy(rows_v, xs_hbm.at[idx_v], sem).wait()


def _gemm_body(bs_ref, xs_hbm, wgu_hbm, wd_hbm, out_hbm,
               xse_v, oute_v, wg_b, wu_b, wd_b, wsem, xsem, osem,
               *, bm, bn, nj):
    e = pl.program_id(0)
    E = wgu_hbm.shape[0]
    D = wd_hbm.shape[1]
    DFF = wd_hbm.shape[2]

    def w_copies(ee, jj, slot):
        return (
            pltpu.make_async_copy(
                wgu_hbm.at[ee, pl.ds(jj * bn, bn), :], wg_b.at[slot],
                wsem.at[slot]),
            pltpu.make_async_copy(
                wgu_hbm.at[ee, pl.ds(DFF + jj * bn, bn), :], wu_b.at[slot],
                wsem.at[slot]),
            pltpu.make_async_copy(
                wd_hbm.at[ee, :, pl.ds(jj * bn, bn)], wd_b.at[slot],
                wsem.at[slot]),
        )

    @pl.when(e == 0)
    def _prologue():
        for c in w_copies(e, 0, 0):
            c.start()

    lo = bs_ref[0, e]
    hi = bs_ref[0, e + 1]

    def x_copy(i):
        return pltpu.make_async_copy(
            xs_hbm.at[pl.ds(i * bm, bm), :],
            xse_v.at[pl.ds((i - lo) * bm, bm), :], xsem)

    def _x_start(i, _):
        x_copy(i).start()
        return 0

    def _x_wait(i, _):
        x_copy(i).wait()
        return 0

    lax.fori_loop(lo, hi, _x_start, 0)
    lax.fori_loop(lo, hi, _x_wait, 0)

    for j in range(nj):
        if j + 1 < nj:
            for c in w_copies(e, j + 1, (j + 1) % 2):
                c.start()
        else:
            @pl.when(e + 1 < E)
            def _prefetch_next_e():
                for c in w_copies(e + 1, 0, 0):
                    c.start()
        for c in w_copies(e, j, j % 2):
            c.wait()
        wg = wg_b[j % 2]
        wu = wu_b[j % 2]
        wd = wd_b[j % 2]

        def blk(i, _):
            lrow = (i - lo) * bm
            xb = xse_v[pl.ds(lrow, bm), :]
            g = lax.dot_general(xb, wg, (((1,), (1,)), ((), ())),
                                preferred_element_type=jnp.float32)
            u = lax.dot_general(xb, wu, (((1,), (1,)), ((), ())),
                                preferred_element_type=jnp.float32)
            h = g * lax.logistic(g) * u
            y = lax.dot_general(h, wd, (((1,), (1,)), ((), ())),
                                preferred_element_type=jnp.float32)
            if j == 0:
                oute_v[pl.ds(lrow, bm), :] = y
            else:
                oute_v[pl.ds(lrow, bm), :] += y
            return 0

        lax.fori_loop(lo, hi, blk, 0)

    def out_copy(i, _):
        cp = pltpu.make_async_copy(
            oute_v.at[pl.ds((i - lo) * bm, bm), :],
            out_hbm.at[pl.ds(i * bm, bm), :], osem)
        cp.start()
        cp.wait()
        return 0

    lax.fori_loop(lo, hi, out_copy, 0)


def _sc_combine_body(ys_hbm, pos1_hbm, pos2_hbm, w1_hbm, w2_hbm, out_hbm,
                     a_v, b_v, idx_v, w1_v, w2_v, sem):
    wid = lax.axis_index("s") * NC + lax.axis_index("c")
    T = out_hbm.shape[0]
    D = out_hbm.shape[1]
    tpw = T // NW
    CHT = 32
    for ci in range(tpw // CHT):
        base = wid * tpw + ci * CHT
        pltpu.sync_copy(pos1_hbm.at[pl.ds(base, CHT)], idx_v)
        pltpu.async_copy(ys_hbm.at[idx_v], a_v, sem).wait()
        pltpu.sync_copy(pos2_hbm.at[pl.ds(base, CHT)], idx_v)
        pltpu.async_copy(ys_hbm.at[idx_v], b_v, sem).wait()
        pltpu.sync_copy(w1_hbm.at[pl.ds(base, CHT)], w1_v)
        pltpu.sync_copy(w2_hbm.at[pl.ds(base, CHT)], w2_v)

        def row_body(r, _):
            s1 = w1_v[r, :]
            s2 = w2_v[r, :]
            for cj in range(D // 16):
                sl = pl.ds(cj * 16, 16)
                a_v[r, sl] = a_v[r, sl] * s1 + b_v[r, sl] * s2
            return 0

        lax.fori_loop(0, CHT, row_body, 0)
        pltpu.sync_copy(a_v, out_hbm.at[pl.ds(base, CHT)])


def kernel(x, gate_w, gate_up_w, down_w):
    B, S, D = x.shape
    T = B * S
    E = gate_w.shape[0]
    DFF = down_w.shape[2]
    K = 2
    BM = 512
    BN = 1024
    NM = (T * K) // BM + E
    MPAD = NM * BM
    J = DFF // BN
    JG = DFF // BN
    x_flat = x.reshape(T, D)

    pos1, pos2, w1, w2, bs, aux = pl.pallas_call(
        functools.partial(_routing_body, bm=BM),
        out_shape=(
            jax.ShapeDtypeStruct((T, 1), jnp.int32),
            jax.ShapeDtypeStruct((T, 1), jnp.int32),
            jax.ShapeDtypeStruct((T, 16), jnp.float32),
            jax.ShapeDtypeStruct((T, 16), jnp.float32),
            jax.ShapeDtypeStruct((1, E + 1), jnp.int32),
            jax.ShapeDtypeStruct((1, 1), jnp.float32),
        ),
    )(x_flat, gate_w)
    pos1 = pos1.reshape(T)
    pos2 = pos2.reshape(T)

    mesh = plsc.VectorSubcoreMesh(core_axis_name="c", subcore_axis_name="s")
    xs = pl.kernel(
        _sc_scatter_body,
        out_type=jax.ShapeDtypeStruct((MPAD, D), jnp.float32),
        mesh=mesh,
        scratch_types=[
            pltpu.VMEM((T // NW, D), jnp.float32),
            pltpu.VMEM((T // NW,), jnp.int32),
            pltpu.SemaphoreType.DMA,
        ],
    )(x_flat, pos1, pos2)

    ys = pl.pallas_call(
        functools.partial(_gemm_body, bm=BM, bn=BN, nj=J),
        grid=(E,),
        in_specs=[
            pl.BlockSpec(memory_space=pltpu.SMEM),
            pl.BlockSpec(memory_space=pl.ANY),
            pl.BlockSpec(memory_space=pl.ANY),
            pl.BlockSpec(memory_space=pl.ANY),
        ],
        out_specs=pl.BlockSpec(memory_space=pl.ANY),
        out_shape=jax.ShapeDtypeStruct((MPAD, D), jnp.float32),
        scratch_shapes=[
            pltpu.VMEM((T, D), jnp.float32),
            pltpu.VMEM((T, D), jnp.float32),
            pltpu.VMEM((2, BN, D), jnp.float32),
            pltpu.VMEM((2, BN, D), jnp.float32),
            pltpu.VMEM((2, D, BN), jnp.float32),
            pltpu.SemaphoreType.DMA((2,)),
            pltpu.SemaphoreType.DMA,
            pltpu.SemaphoreType.DMA,
        ],
    )(bs, xs, gate_up_w, down_w)

    out = pl.kernel(
        _sc_combine_body,
        out_type=jax.ShapeDtypeStruct((T, D), jnp.float32),
        mesh=mesh,
        scratch_types=[
            pltpu.VMEM((32, D), jnp.float32),
            pltpu.VMEM((32, D), jnp.float32),
            pltpu.VMEM((32,), jnp.int32),
            pltpu.VMEM((32, 16), jnp.float32),
            pltpu.VMEM((32, 16), jnp.float32),
            pltpu.SemaphoreType.DMA,
        ],
    )(ys, pos1, pos2, w1, w2)

    return out.reshape(B, S, D), aux.reshape(())
